# baseline (speedup 1.0000x reference)
import functools
import math

import jax
import jax.numpy as jnp
from jax import lax
from jax.experimental import pallas as pl
from jax.experimental.pallas import tpu as pltpu
from jax.experimental.pallas import tpu_sc as plsc

_EMBED = 64
_LANES = 16
_SCALE = math.sqrt(_EMBED)
_PAIR_W = 2 * _EMBED

_info = plsc.get_sparse_core_info()
_NC, _NS = _info.num_cores, _info.num_subcores
_NW = _NC * _NS
_NBUF = 2


def _make_lookup(src_len: int, batch: int, vocab: int):
    cols_per_w = batch // _NW
    mesh = plsc.VectorSubcoreMesh(core_axis_name="c", subcore_axis_name="s")

    @functools.partial(
        pl.kernel,
        out_type=jax.ShapeDtypeStruct((src_len, _EMBED, batch), jnp.float32),
        mesh=mesh,
        scratch_types=[
            pltpu.VMEM((src_len, cols_per_w), jnp.int32),
            [pltpu.VMEM((cols_per_w, _PAIR_W), jnp.float32) for _ in range(_NBUF)],
            [pltpu.VMEM((_EMBED, cols_per_w), jnp.float32) for _ in range(_NBUF)],
            [pltpu.SemaphoreType.DMA for _ in range(_NBUF)],
            [pltpu.SemaphoreType.DMA for _ in range(_NBUF)],
        ],
        compiler_params=pltpu.CompilerParams(needs_layout_passes=False),
    )
    def lookup(tok_hbm, wide_hbm, out_hbm, idx_v, row_bufs, stage_bufs, gsems, ssems):
        w = lax.axis_index("s") * _NC + lax.axis_index("c")
        col0 = w * cols_per_w
        pltpu.sync_copy(tok_hbm.at[:, pl.ds(col0, cols_per_w)], idx_v)

        def gather_desc(s, b):
            return pltpu.make_async_copy(
                wide_hbm.at[idx_v.at[s]], row_bufs[b], gsems[b]
            )

        def out_desc(s, b):
            return pltpu.make_async_copy(
                stage_bufs[b], out_hbm.at[s, :, pl.ds(col0, cols_per_w)], ssems[b]
            )

        def transpose_scale(b):
            @pl.loop(0, _EMBED)
            def _comp(e):
                e_vec = lax.broadcast_in_dim(e, (_LANES,), ())
                for l0 in range(cols_per_w // _LANES):
                    rows = lax.iota(jnp.int32, _LANES) + (l0 * _LANES)
                    v = plsc.load_gather(row_bufs[b], [rows, e_vec])
                    stage_bufs[b][e, pl.ds(l0 * _LANES, _LANES)] = v * _SCALE

        for b in range(_NBUF):
            gather_desc(b, b).start()

        @pl.loop(0, src_len, step=_NBUF)
        def _pipeline(c0):
            for b in range(_NBUF):
                s = c0 + b
                gather_desc(s, b).wait()

                @pl.when(c0 >= _NBUF)
                def _():
                    out_desc(s - _NBUF, b).wait()

                transpose_scale(b)
                out_desc(s, b).start()

                @pl.when(c0 < src_len - _NBUF)
                def _():
                    gather_desc(s + _NBUF, b).start()

        for b in range(_NBUF):
            out_desc(src_len - _NBUF + b, b).wait()

    return lookup


def kernel(tokens, table):
    src_len, batch = tokens.shape
    vocab, embed = table.shape
    wide = jnp.pad(table, ((0, 0), (0, _PAIR_W - embed)))
    out_t = _make_lookup(src_len, batch, vocab)(tokens.astype(jnp.int32), wide)
    return out_t.transpose(0, 2, 1)


# speedup vs baseline: 1.4544x; 1.4544x over previous
import functools
import math

import jax
import jax.numpy as jnp
from jax import lax
from jax.experimental import pallas as pl
from jax.experimental.pallas import tpu as pltpu
from jax.experimental.pallas import tpu_sc as plsc

_EMBED = 64
_LANES = 16
_VPR = _EMBED // _LANES
_SCALE = math.sqrt(_EMBED)
_PAIR_W = 2 * _EMBED

_info = plsc.get_sparse_core_info()
_NC, _NS = _info.num_cores, _info.num_subcores
_NW = _NC * _NS
_NBUF = 2


def _make_lookup(src_len: int, batch: int, vocab: int):
    cols_per_w = batch // _NW
    mesh = plsc.VectorSubcoreMesh(core_axis_name="c", subcore_axis_name="s")

    @functools.partial(
        pl.kernel,
        out_type=jax.ShapeDtypeStruct((src_len, batch, _EMBED), jnp.float32),
        mesh=mesh,
        scratch_types=[
            pltpu.VMEM((src_len, cols_per_w), jnp.int32),
            [pltpu.VMEM((cols_per_w, _PAIR_W), jnp.float32) for _ in range(_NBUF)],
            [pltpu.VMEM((cols_per_w, _EMBED), jnp.float32) for _ in range(_NBUF)],
            [pltpu.SemaphoreType.DMA for _ in range(_NBUF)],
            [pltpu.SemaphoreType.DMA for _ in range(_NBUF)],
        ],
        compiler_params=pltpu.CompilerParams(disable_bounds_checks=True),
    )
    def lookup(tok_hbm, wide_hbm, out_hbm, idx_v, row_bufs, stage_bufs, gsems, ssems):
        w = lax.axis_index("s") * _NC + lax.axis_index("c")
        col0 = w * cols_per_w
        pltpu.sync_copy(tok_hbm.at[:, pl.ds(col0, cols_per_w)], idx_v)

        def gather_desc(s, b):
            return pltpu.make_async_copy(
                wide_hbm.at[idx_v.at[s]], row_bufs[b], gsems[b]
            )

        def out_desc(s, b):
            return pltpu.make_async_copy(
                stage_bufs[b], out_hbm.at[s, pl.ds(col0, cols_per_w)], ssems[b]
            )

        def scale(b):
            @pl.loop(0, cols_per_w, unroll=4)
            def _row(r):
                for j in range(_VPR):
                    sl = pl.ds(j * _LANES, _LANES)
                    stage_bufs[b][r, sl] = row_bufs[b][r, sl] * _SCALE

        for b in range(_NBUF):
            gather_desc(b, b).start()

        @pl.loop(0, src_len, step=_NBUF)
        def _pipeline(c0):
            for b in range(_NBUF):
                s = c0 + b
                gather_desc(s, b).wait()

                @pl.when(c0 >= _NBUF)
                def _():
                    out_desc(s - _NBUF, b).wait()

                scale(b)
                out_desc(s, b).start()

                @pl.when(c0 < src_len - _NBUF)
                def _():
                    gather_desc(s + _NBUF, b).start()

        for b in range(_NBUF):
            out_desc(src_len - _NBUF + b, b).wait()

    return lookup


def kernel(tokens, table):
    src_len, batch = tokens.shape
    vocab, embed = table.shape
    wide = jnp.pad(table, ((0, 0), (0, _PAIR_W - embed)))
    return _make_lookup(src_len, batch, vocab)(tokens.astype(jnp.int32), wide)


# speedup vs baseline: 1.6964x; 1.1664x over previous
import functools
import math

import jax
import jax.numpy as jnp
from jax import lax
from jax.experimental import pallas as pl
from jax.experimental.pallas import tpu as pltpu
from jax.experimental.pallas import tpu_sc as plsc

_EMBED = 64
_LANES = 16
_VPR = _EMBED // _LANES
_SCALE = math.sqrt(_EMBED)
_PAIR_W = 2 * _EMBED

_info = plsc.get_sparse_core_info()
_NC, _NS = _info.num_cores, _info.num_subcores
_NW = _NC * _NS
_NBUF = 2


def _make_lookup(src_len: int, batch: int, vocab: int):
    cols_per_w = batch // _NW
    mesh = plsc.VectorSubcoreMesh(core_axis_name="c", subcore_axis_name="s")

    @functools.partial(
        pl.kernel,
        out_type=jax.ShapeDtypeStruct((src_len, batch, _EMBED), jnp.float32),
        mesh=mesh,
        scratch_types=[
            pltpu.VMEM((src_len, cols_per_w), jnp.int32),
            [pltpu.VMEM((cols_per_w, _PAIR_W), jnp.float32) for _ in range(_NBUF)],
            [pltpu.VMEM((cols_per_w, _EMBED), jnp.float32) for _ in range(_NBUF)],
            [pltpu.SemaphoreType.DMA for _ in range(_NBUF)],
            [pltpu.SemaphoreType.DMA for _ in range(_NBUF)],
        ],
    )
    def lookup(tok_hbm, wide_hbm, out_hbm, idx_v, row_bufs, stage_bufs, gsems, ssems):
        w = lax.axis_index("s") * _NC + lax.axis_index("c")
        col0 = w * cols_per_w
        pltpu.sync_copy(tok_hbm.at[:, pl.ds(col0, cols_per_w)], idx_v)

        def gather_desc(s, b):
            return pltpu.make_async_copy(
                wide_hbm.at[idx_v.at[s]], row_bufs[b], gsems[b]
            )

        def out_desc(s, b):
            return pltpu.make_async_copy(
                stage_bufs[b], out_hbm.at[s, pl.ds(col0, cols_per_w)], ssems[b]
            )

        def scale(b):
            @pl.loop(0, cols_per_w)
            def _row(r):
                for j in range(_VPR):
                    sl = pl.ds(j * _LANES, _LANES)
                    stage_bufs[b][r, sl] = row_bufs[b][r, sl] * _SCALE

        for b in range(_NBUF):
            gather_desc(b, b).start()

        @pl.loop(0, src_len, step=_NBUF)
        def _pipeline(c0):
            for b in range(_NBUF):
                s = c0 + b
                gather_desc(s, b).wait()

                @pl.when(c0 >= _NBUF)
                def _():
                    out_desc(s - _NBUF, b).wait()

                scale(b)
                out_desc(s, b).start()

                @pl.when(c0 < src_len - _NBUF)
                def _():
                    gather_desc(s + _NBUF, b).start()

        for b in range(_NBUF):
            out_desc(src_len - _NBUF + b, b).wait()

    return lookup


def kernel(tokens, table):
    src_len, batch = tokens.shape
    vocab, embed = table.shape
    wide = jnp.pad(table, ((0, 0), (0, _PAIR_W - embed)))
    return _make_lookup(src_len, batch, vocab)(tokens.astype(jnp.int32), wide)
